# Initial kernel scaffold; baseline (speedup 1.0000x reference)
import jax
import jax.numpy as jnp
from jax.experimental import pallas as pl


def kernel(x, batch, c1_W1, c1_b1, c1_g1, c1_be1, c1_W2, c1_b2, c2_W1, c2_b1, c2_g1, c2_be1, c2_W2, c2_b2, c3_W1, c3_b1, c3_g1, c3_be1, c3_W2, c3_b2, m_W1, m_b1, m_W2, m_b2, m_W3, m_b3, m_W4, m_b4):
    raise NotImplementedError("write your pallas kernel here")


# speedup vs baseline: 1.8355x; 1.8355x over previous
import functools

import jax
import jax.numpy as jnp
from jax import lax
from jax.experimental import pallas as pl
from jax.experimental.pallas import tpu as pltpu
from jax.experimental.pallas import tpu_sc as plsc

_N = 4096
_K = 16
_NK = _N * _K
_EPS = 1e-5
_DH = 64
_ROWS = 256
_NW = 32
_CH = 128


def _knn_pq_body(xf_ref, xaT_ref, brow_ref, bcol_ref, wp_ref, wb_ref, b1_ref,
                 idx_ref, p_ref, q_ref):
    xb = xf_ref[...]
    xaT = xaT_ref[...]
    prod = lax.dot_general(xb, xaT, (((1,), (0,)), ((), ())),
                           preferred_element_type=jnp.float32)
    sqb = jnp.sum(xb * xb, axis=1, keepdims=True)
    sqa = jnp.sum(xaT * xaT, axis=0, keepdims=True)
    d = (sqb + sqa) - 2.0 * prod
    d = jnp.where(brow_ref[...] != bcol_ref[...], jnp.inf, d)
    iota = lax.broadcasted_iota(jnp.int32, d.shape, 1)
    sel = jnp.zeros(d.shape, jnp.bool_)
    kiota = lax.broadcasted_iota(jnp.int32, (xb.shape[0], _K), 1)
    idx_acc = jnp.zeros((xb.shape[0], _K), jnp.int32)
    for k in range(_K):
        dv = jnp.where(sel, jnp.inf, d)
        m = jnp.min(dv, axis=1, keepdims=True)
        cand = jnp.where((dv == m) & (~sel), iota, _N)
        amin = jnp.min(cand, axis=1, keepdims=True)
        sel = sel | (iota == amin)
        idx_acc = jnp.where(kiota == k, amin, idx_acc)
    idx_ref[...] = idx_acc
    p_ref[...] = xb @ wp_ref[...] + b1_ref[...]
    q_ref[...] = xb @ wb_ref[...]


def _stats_body(qg_ref, p_ref, acc_ref):
    h = p_ref[...][:, None, :] + qg_ref[...]
    s = jnp.sum(jnp.sum(h, axis=1), axis=0, keepdims=True)
    s2 = jnp.sum(jnp.sum(h * h, axis=1), axis=0, keepdims=True)
    val = jnp.concatenate([s, s2], axis=0)
    @pl.when(pl.program_id(0) == 0)
    def _():
        acc_ref[...] = val
    @pl.when(pl.program_id(0) != 0)
    def _():
        acc_ref[...] = acc_ref[...] + val


def _edge_out_body(qg_ref, p_ref, acc_ref, g_ref, be_ref, w2_ref, b2_ref,
                   o_ref):
    inv_n = 1.0 / float(_NK)
    s = acc_ref[0:1, :]
    s2 = acc_ref[1:2, :]
    mean = s * inv_n
    var = s2 * inv_n - mean * mean
    a = g_ref[...] * lax.rsqrt(var + _EPS)
    c = be_ref[...] - a * mean
    h = p_ref[...][:, None, :] + qg_ref[...]
    hn = jnp.maximum(h * a[:, None, :] + c[:, None, :], 0.0)
    r = hn.shape[0]
    y = hn.reshape(r * _K, _DH) @ w2_ref[...] + b2_ref[...]
    o_ref[...] = jnp.max(y.reshape(r, _K, _DH), axis=1)


def _mlp_body(xc_ref, w1_ref, b1_ref, w2_ref, b2_ref, w3_ref, b3_ref,
              w4_ref, b4_ref, o_ref):
    h = xc_ref[...]
    h = jnp.maximum(h @ w1_ref[...] + b1_ref[...], 0.0)
    h = jnp.maximum(h @ w2_ref[...] + b2_ref[...], 0.0)
    h = jnp.maximum(h @ w3_ref[...] + b3_ref[...], 0.0)
    h = h @ w4_ref[...] + b4_ref[...]
    z = h - jnp.max(h, axis=1, keepdims=True)
    o_ref[...] = z - jnp.log(jnp.sum(jnp.exp(z), axis=1, keepdims=True))


def _gather_rows(q, idx_flat):
    per_w = _NK // _NW
    mesh = plsc.VectorSubcoreMesh(core_axis_name="c", subcore_axis_name="s")

    @functools.partial(
        pl.kernel,
        out_type=jax.ShapeDtypeStruct((_NK, _DH), jnp.float32),
        mesh=mesh,
        scratch_types=[
            pltpu.VMEM((_CH,), jnp.int32),
            pltpu.VMEM((_CH, _DH), jnp.float32),
            pltpu.SemaphoreType.DMA,
        ],
        compiler_params=pltpu.CompilerParams(use_tc_tiling_on_sc=False),
    )
    def gk(q_hbm, idx_hbm, out_hbm, idx_v, rows_v, sem):
        w = lax.axis_index("s") * 2 + lax.axis_index("c")
        base = w * per_w
        for cc in range(per_w // _CH):
            off = base + cc * _CH
            pltpu.sync_copy(idx_hbm.at[pl.ds(off, _CH)], idx_v)
            pltpu.async_copy(q_hbm.at[idx_v], rows_v, sem).wait()
            pltpu.sync_copy(rows_v, out_hbm.at[pl.ds(off, _CH)])

    return gk(q, idx_flat)


def _edge_conv(xf, xaT, brow, bcol, wp, wb, b1, g1, be1, w2, b2):
    f = xf.shape[1]
    grid = (_N // _ROWS,)
    idx, p, q = pl.pallas_call(
        _knn_pq_body,
        grid=grid,
        in_specs=[
            pl.BlockSpec((_ROWS, f), lambda i: (i, 0)),
            pl.BlockSpec((f, _N), lambda i: (0, 0)),
            pl.BlockSpec((_ROWS, 1), lambda i: (i, 0)),
            pl.BlockSpec((1, _N), lambda i: (0, 0)),
            pl.BlockSpec((f, _DH), lambda i: (0, 0)),
            pl.BlockSpec((f, _DH), lambda i: (0, 0)),
            pl.BlockSpec((1, _DH), lambda i: (0, 0)),
        ],
        out_specs=[
            pl.BlockSpec((_ROWS, _K), lambda i: (i, 0)),
            pl.BlockSpec((_ROWS, _DH), lambda i: (i, 0)),
            pl.BlockSpec((_ROWS, _DH), lambda i: (i, 0)),
        ],
        out_shape=[
            jax.ShapeDtypeStruct((_N, _K), jnp.int32),
            jax.ShapeDtypeStruct((_N, _DH), jnp.float32),
            jax.ShapeDtypeStruct((_N, _DH), jnp.float32),
        ],
    )(xf, xaT, brow, bcol, wp, wb, b1)

    qg = _gather_rows(q, idx.reshape(_NK))
    qg3 = qg.reshape(_N, _K, _DH)

    acc = pl.pallas_call(
        _stats_body,
        grid=grid,
        in_specs=[
            pl.BlockSpec((_ROWS, _K, _DH), lambda i: (i, 0, 0)),
            pl.BlockSpec((_ROWS, _DH), lambda i: (i, 0)),
        ],
        out_specs=pl.BlockSpec((2, _DH), lambda i: (0, 0)),
        out_shape=jax.ShapeDtypeStruct((2, _DH), jnp.float32),
        compiler_params=pltpu.CompilerParams(
            dimension_semantics=("arbitrary",)),
    )(qg3, p)

    xo = pl.pallas_call(
        _edge_out_body,
        grid=grid,
        in_specs=[
            pl.BlockSpec((_ROWS, _K, _DH), lambda i: (i, 0, 0)),
            pl.BlockSpec((_ROWS, _DH), lambda i: (i, 0)),
            pl.BlockSpec((2, _DH), lambda i: (0, 0)),
            pl.BlockSpec((1, _DH), lambda i: (0, 0)),
            pl.BlockSpec((1, _DH), lambda i: (0, 0)),
            pl.BlockSpec((_DH, _DH), lambda i: (0, 0)),
            pl.BlockSpec((1, _DH), lambda i: (0, 0)),
        ],
        out_specs=pl.BlockSpec((_ROWS, _DH), lambda i: (i, 0)),
        out_shape=jax.ShapeDtypeStruct((_N, _DH), jnp.float32),
    )(qg3, p, acc, g1, be1, w2, b2)
    return xo


def kernel(x, batch, c1_W1, c1_b1, c1_g1, c1_be1, c1_W2, c1_b2,
           c2_W1, c2_b1, c2_g1, c2_be1, c2_W2, c2_b2,
           c3_W1, c3_b1, c3_g1, c3_be1, c3_W2, c3_b2,
           m_W1, m_b1, m_W2, m_b2, m_W3, m_b3, m_W4, m_b4):
    batch = batch.astype(jnp.int32)
    brow = batch.reshape(_N, 1)
    bcol = batch.reshape(1, _N)

    xf1 = jnp.pad(x, ((0, 0), (0, 7)))
    wp1 = jnp.pad(c1_W1[0:1] - c1_W1[1:2], ((0, 7), (0, 0)))
    wb1 = jnp.pad(c1_W1[1:2], ((0, 7), (0, 0)))
    x1 = _edge_conv(xf1, xf1.T, brow, bcol, wp1, wb1,
                    c1_b1.reshape(1, _DH), c1_g1.reshape(1, _DH),
                    c1_be1.reshape(1, _DH), c1_W2, c1_b2.reshape(1, _DH))

    x2 = _edge_conv(x1, x1.T, brow, bcol,
                    c2_W1[:_DH] - c2_W1[_DH:], c2_W1[_DH:],
                    c2_b1.reshape(1, _DH), c2_g1.reshape(1, _DH),
                    c2_be1.reshape(1, _DH), c2_W2, c2_b2.reshape(1, _DH))

    x3 = _edge_conv(x2, x2.T, brow, bcol,
                    c3_W1[:_DH] - c3_W1[_DH:], c3_W1[_DH:],
                    c3_b1.reshape(1, _DH), c3_g1.reshape(1, _DH),
                    c3_be1.reshape(1, _DH), c3_W2, c3_b2.reshape(1, _DH))

    xc = jnp.concatenate([x1, x2, x3], axis=1)
    rb = 1024
    out = pl.pallas_call(
        _mlp_body,
        grid=(_N // rb,),
        in_specs=[pl.BlockSpec((rb, 192), lambda i: (i, 0))] + [
            pl.BlockSpec(s, lambda i: (0, 0)) for s in
            [(192, 256), (1, 256), (256, 128), (1, 128),
             (128, 64), (1, 64), (64, 16), (1, 16)]
        ],
        out_specs=pl.BlockSpec((rb, 16), lambda i: (i, 0)),
        out_shape=jax.ShapeDtypeStruct((_N, 16), jnp.float32),
    )(xc, m_W1, m_b1.reshape(1, 256), m_W2, m_b2.reshape(1, 128),
      m_W3, m_b3.reshape(1, 64), m_W4, m_b4.reshape(1, 16))
    return out


# speedup vs baseline: 4.1462x; 2.2588x over previous
import functools

import jax
import jax.numpy as jnp
from jax import lax
from jax.experimental import pallas as pl
from jax.experimental.pallas import tpu as pltpu
from jax.experimental.pallas import tpu_sc as plsc

_N = 4096
_K = 16
_NK = _N * _K
_EPS = 1e-5
_DH = 64
_ROWS = 256
_NW = 32
_CH = 128


def _knn_pq_body(xf_ref, xaT_ref, brow_ref, bcol_ref, wp_ref, wb_ref, b1_ref,
                 idx_ref, p_ref, q_ref):
    xb = xf_ref[...]
    xaT = xaT_ref[...]
    prod = lax.dot_general(xb, xaT, (((1,), (0,)), ((), ())),
                           preferred_element_type=jnp.float32)
    sqb = jnp.sum(xb * xb, axis=1, keepdims=True)
    sqa = jnp.sum(xaT * xaT, axis=0, keepdims=True)
    d = (sqb + sqa) - 2.0 * prod
    d = jnp.where(brow_ref[...] != bcol_ref[...], jnp.inf, d)
    d = jnp.maximum(d, 0.0)
    iota = lax.broadcasted_iota(jnp.int32, d.shape, 1)
    key = (lax.bitcast_convert_type(d, jnp.int32) & jnp.int32(~0xFFF)) | iota
    kiota = lax.broadcasted_iota(jnp.int32, (xb.shape[0], _K), 1)
    idx_acc = jnp.zeros((xb.shape[0], _K), jnp.int32)
    for k in range(_K):
        m = jnp.min(key, axis=1, keepdims=True)
        key = jnp.where(key == m, jnp.int32(0x7FFFFFFF), key)
        idx_acc = jnp.where(kiota == k, m & 0xFFF, idx_acc)
    idx_ref[...] = idx_acc
    p_ref[...] = xb @ wp_ref[...] + b1_ref[...]
    q_ref[...] = xb @ wb_ref[...]


def _stats_body(qg_ref, p_ref, acc_ref):
    h = p_ref[...][:, None, :] + qg_ref[...]
    s = jnp.sum(jnp.sum(h, axis=1), axis=0, keepdims=True)
    s2 = jnp.sum(jnp.sum(h * h, axis=1), axis=0, keepdims=True)
    val = jnp.concatenate([s, s2], axis=0)
    @pl.when(pl.program_id(0) == 0)
    def _():
        acc_ref[...] = val
    @pl.when(pl.program_id(0) != 0)
    def _():
        acc_ref[...] = acc_ref[...] + val


def _edge_out_body(qg_ref, p_ref, acc_ref, g_ref, be_ref, w2_ref, b2_ref,
                   o_ref):
    inv_n = 1.0 / float(_NK)
    s = acc_ref[0:1, :]
    s2 = acc_ref[1:2, :]
    mean = s * inv_n
    var = s2 * inv_n - mean * mean
    a = g_ref[...] * lax.rsqrt(var + _EPS)
    c = be_ref[...] - a * mean
    h = p_ref[...][:, None, :] + qg_ref[...]
    hn = jnp.maximum(h * a[:, None, :] + c[:, None, :], 0.0)
    r = hn.shape[0]
    y = hn.reshape(r * _K, _DH) @ w2_ref[...] + b2_ref[...]
    o_ref[...] = jnp.max(y.reshape(r, _K, _DH), axis=1)


def _mlp_body(xc_ref, w1_ref, b1_ref, w2_ref, b2_ref, w3_ref, b3_ref,
              w4_ref, b4_ref, o_ref):
    h = xc_ref[...]
    h = jnp.maximum(h @ w1_ref[...] + b1_ref[...], 0.0)
    h = jnp.maximum(h @ w2_ref[...] + b2_ref[...], 0.0)
    h = jnp.maximum(h @ w3_ref[...] + b3_ref[...], 0.0)
    h = h @ w4_ref[...] + b4_ref[...]
    z = h - jnp.max(h, axis=1, keepdims=True)
    o_ref[...] = z - jnp.log(jnp.sum(jnp.exp(z), axis=1, keepdims=True))


def _gather_rows(q, idx_flat):
    per_w = _NK // _NW
    mesh = plsc.VectorSubcoreMesh(core_axis_name="c", subcore_axis_name="s")

    @functools.partial(
        pl.kernel,
        out_type=jax.ShapeDtypeStruct((_NK, _DH), jnp.float32),
        mesh=mesh,
        scratch_types=[
            pltpu.VMEM((_CH,), jnp.int32),
            pltpu.VMEM((_CH, _DH), jnp.float32),
            pltpu.SemaphoreType.DMA,
        ],
        compiler_params=pltpu.CompilerParams(use_tc_tiling_on_sc=False),
    )
    def gk(q_hbm, idx_hbm, out_hbm, idx_v, rows_v, sem):
        w = lax.axis_index("s") * 2 + lax.axis_index("c")
        base = w * per_w
        for cc in range(per_w // _CH):
            off = base + cc * _CH
            pltpu.sync_copy(idx_hbm.at[pl.ds(off, _CH)], idx_v)
            pltpu.async_copy(q_hbm.at[idx_v], rows_v, sem).wait()
            pltpu.sync_copy(rows_v, out_hbm.at[pl.ds(off, _CH)])

    return gk(q, idx_flat)


def _edge_conv(xf, xaT, brow, bcol, wp, wb, b1, g1, be1, w2, b2):
    f = xf.shape[1]
    grid = (_N // _ROWS,)
    idx, p, q = pl.pallas_call(
        _knn_pq_body,
        grid=grid,
        in_specs=[
            pl.BlockSpec((_ROWS, f), lambda i: (i, 0)),
            pl.BlockSpec((f, _N), lambda i: (0, 0)),
            pl.BlockSpec((_ROWS, 1), lambda i: (i, 0)),
            pl.BlockSpec((1, _N), lambda i: (0, 0)),
            pl.BlockSpec((f, _DH), lambda i: (0, 0)),
            pl.BlockSpec((f, _DH), lambda i: (0, 0)),
            pl.BlockSpec((1, _DH), lambda i: (0, 0)),
        ],
        out_specs=[
            pl.BlockSpec((_ROWS, _K), lambda i: (i, 0)),
            pl.BlockSpec((_ROWS, _DH), lambda i: (i, 0)),
            pl.BlockSpec((_ROWS, _DH), lambda i: (i, 0)),
        ],
        out_shape=[
            jax.ShapeDtypeStruct((_N, _K), jnp.int32),
            jax.ShapeDtypeStruct((_N, _DH), jnp.float32),
            jax.ShapeDtypeStruct((_N, _DH), jnp.float32),
        ],
    )(xf, xaT, brow, bcol, wp, wb, b1)

    qg = _gather_rows(q, idx.reshape(_NK))
    qg3 = qg.reshape(_N, _K, _DH)

    acc = pl.pallas_call(
        _stats_body,
        grid=grid,
        in_specs=[
            pl.BlockSpec((_ROWS, _K, _DH), lambda i: (i, 0, 0)),
            pl.BlockSpec((_ROWS, _DH), lambda i: (i, 0)),
        ],
        out_specs=pl.BlockSpec((2, _DH), lambda i: (0, 0)),
        out_shape=jax.ShapeDtypeStruct((2, _DH), jnp.float32),
        compiler_params=pltpu.CompilerParams(
            dimension_semantics=("arbitrary",)),
    )(qg3, p)

    xo = pl.pallas_call(
        _edge_out_body,
        grid=grid,
        in_specs=[
            pl.BlockSpec((_ROWS, _K, _DH), lambda i: (i, 0, 0)),
            pl.BlockSpec((_ROWS, _DH), lambda i: (i, 0)),
            pl.BlockSpec((2, _DH), lambda i: (0, 0)),
            pl.BlockSpec((1, _DH), lambda i: (0, 0)),
            pl.BlockSpec((1, _DH), lambda i: (0, 0)),
            pl.BlockSpec((_DH, _DH), lambda i: (0, 0)),
            pl.BlockSpec((1, _DH), lambda i: (0, 0)),
        ],
        out_specs=pl.BlockSpec((_ROWS, _DH), lambda i: (i, 0)),
        out_shape=jax.ShapeDtypeStruct((_N, _DH), jnp.float32),
    )(qg3, p, acc, g1, be1, w2, b2)
    return xo


def kernel(x, batch, c1_W1, c1_b1, c1_g1, c1_be1, c1_W2, c1_b2,
           c2_W1, c2_b1, c2_g1, c2_be1, c2_W2, c2_b2,
           c3_W1, c3_b1, c3_g1, c3_be1, c3_W2, c3_b2,
           m_W1, m_b1, m_W2, m_b2, m_W3, m_b3, m_W4, m_b4):
    batch = batch.astype(jnp.int32)
    brow = batch.reshape(_N, 1)
    bcol = batch.reshape(1, _N)

    xf1 = jnp.pad(x, ((0, 0), (0, 7)))
    wp1 = jnp.pad(c1_W1[0:1] - c1_W1[1:2], ((0, 7), (0, 0)))
    wb1 = jnp.pad(c1_W1[1:2], ((0, 7), (0, 0)))
    x1 = _edge_conv(xf1, xf1.T, brow, bcol, wp1, wb1,
                    c1_b1.reshape(1, _DH), c1_g1.reshape(1, _DH),
                    c1_be1.reshape(1, _DH), c1_W2, c1_b2.reshape(1, _DH))

    x2 = _edge_conv(x1, x1.T, brow, bcol,
                    c2_W1[:_DH] - c2_W1[_DH:], c2_W1[_DH:],
                    c2_b1.reshape(1, _DH), c2_g1.reshape(1, _DH),
                    c2_be1.reshape(1, _DH), c2_W2, c2_b2.reshape(1, _DH))

    x3 = _edge_conv(x2, x2.T, brow, bcol,
                    c3_W1[:_DH] - c3_W1[_DH:], c3_W1[_DH:],
                    c3_b1.reshape(1, _DH), c3_g1.reshape(1, _DH),
                    c3_be1.reshape(1, _DH), c3_W2, c3_b2.reshape(1, _DH))

    xc = jnp.concatenate([x1, x2, x3], axis=1)
    rb = 1024
    out = pl.pallas_call(
        _mlp_body,
        grid=(_N // rb,),
        in_specs=[pl.BlockSpec((rb, 192), lambda i: (i, 0))] + [
            pl.BlockSpec(s, lambda i: (0, 0)) for s in
            [(192, 256), (1, 256), (256, 128), (1, 128),
             (128, 64), (1, 64), (64, 16), (1, 16)]
        ],
        out_specs=pl.BlockSpec((rb, 16), lambda i: (i, 0)),
        out_shape=jax.ShapeDtypeStruct((_N, 16), jnp.float32),
    )(xc, m_W1, m_b1.reshape(1, 256), m_W2, m_b2.reshape(1, 128),
      m_W3, m_b3.reshape(1, 64), m_W4, m_b4.reshape(1, 16))
    return out


# speedup vs baseline: 5.4671x; 1.3186x over previous
import functools

import jax
import jax.numpy as jnp
from jax import lax
from jax.experimental import pallas as pl
from jax.experimental.pallas import tpu as pltpu
from jax.experimental.pallas import tpu_sc as plsc

_N = 4096
_K = 16
_NK = _N * _K
_EPS = 1e-5
_DH = 64
_ROWS = 256
_W = 1536
_NW = 32
_CH = 128


def _topk_from_key(key, r):
    kiota = lax.broadcasted_iota(jnp.int32, (r, _K), 1)
    idx_acc = jnp.zeros((r, _K), jnp.int32)
    for k in range(_K):
        m = jnp.min(key, axis=1, keepdims=True)
        key = jnp.where(key == m, jnp.int32(0x7FFFFFFF), key)
        idx_acc = jnp.where(kiota == k, m & 0xFFF, idx_acc)
    return idx_acc


def _knn_pq_body(xf_ref, xaT_ref, brow_ref, bcol_ref, wp_ref, wb_ref, b1_ref,
                 idx_ref, p_ref, q_ref):
    xb = xf_ref[...]
    r = xb.shape[0]
    sqb = jnp.sum(xb * xb, axis=1, keepdims=True)
    brow = brow_ref[...]

    def compute_key(xaT, bcol, base):
        prod = lax.dot_general(xb, xaT, (((1,), (0,)), ((), ())),
                               preferred_element_type=jnp.float32)
        sqa = jnp.sum(xaT * xaT, axis=0, keepdims=True)
        d = (sqb + sqa) - 2.0 * prod
        d = jnp.where(brow != bcol, jnp.inf, d)
        d = jnp.maximum(d, 0.0)
        iota = lax.broadcasted_iota(jnp.int32, d.shape, 1) + base
        return (lax.bitcast_convert_type(d, jnp.int32)
                & jnp.int32(~0xFFF)) | iota

    b_first = brow_ref[0, 0]
    b_last = brow_ref[r - 1, 0]
    bcol_full = bcol_ref[...]
    citer = lax.broadcasted_iota(jnp.int32, (1, _N), 1)
    lo = jnp.min(jnp.where(bcol_full == b_first, citer, _N))
    hi = jnp.max(jnp.where(bcol_full == b_last, citer, -1)) + 1
    lo_a = jnp.minimum((lo // 128) * 128, _N - _W)

    def wpath(_):
        return _topk_from_key(
            compute_key(xaT_ref[:, pl.ds(lo_a, _W)],
                        bcol_ref[:, pl.ds(lo_a, _W)], lo_a), r)

    def fpath(_):
        return _topk_from_key(compute_key(xaT_ref[...], bcol_full, 0), r)

    idx_ref[...] = lax.cond(hi - lo_a <= _W, wpath, fpath, 0)
    p_ref[...] = xb @ wp_ref[...] + b1_ref[...]
    q_ref[...] = xb @ wb_ref[...]


def _stats_body(qg_ref, p_ref, acc_ref):
    h = p_ref[...][:, None, :] + qg_ref[...]
    s = jnp.sum(jnp.sum(h, axis=1), axis=0, keepdims=True)
    s2 = jnp.sum(jnp.sum(h * h, axis=1), axis=0, keepdims=True)
    val = jnp.concatenate([s, s2], axis=0)
    @pl.when(pl.program_id(0) == 0)
    def _():
        acc_ref[...] = val
    @pl.when(pl.program_id(0) != 0)
    def _():
        acc_ref[...] = acc_ref[...] + val


def _edge_out_body(qg_ref, p_ref, acc_ref, g_ref, be_ref, w2_ref, b2_ref,
                   o_ref):
    inv_n = 1.0 / float(_NK)
    s = acc_ref[0:1, :]
    s2 = acc_ref[1:2, :]
    mean = s * inv_n
    var = s2 * inv_n - mean * mean
    a = g_ref[...] * lax.rsqrt(var + _EPS)
    c = be_ref[...] - a * mean
    h = p_ref[...][:, None, :] + qg_ref[...]
    hn = jnp.maximum(h * a[:, None, :] + c[:, None, :], 0.0)
    r = hn.shape[0]
    y = hn.reshape(r * _K, _DH) @ w2_ref[...] + b2_ref[...]
    o_ref[...] = jnp.max(y.reshape(r, _K, _DH), axis=1)


def _mlp_body(xc_ref, w1_ref, b1_ref, w2_ref, b2_ref, w3_ref, b3_ref,
              w4_ref, b4_ref, o_ref):
    h = xc_ref[...]
    h = jnp.maximum(h @ w1_ref[...] + b1_ref[...], 0.0)
    h = jnp.maximum(h @ w2_ref[...] + b2_ref[...], 0.0)
    h = jnp.maximum(h @ w3_ref[...] + b3_ref[...], 0.0)
    h = h @ w4_ref[...] + b4_ref[...]
    z = h - jnp.max(h, axis=1, keepdims=True)
    o_ref[...] = z - jnp.log(jnp.sum(jnp.exp(z), axis=1, keepdims=True))


def _gather_rows(q, idx_flat):
    per_w = _NK // _NW
    mesh = plsc.VectorSubcoreMesh(core_axis_name="c", subcore_axis_name="s")

    @functools.partial(
        pl.kernel,
        out_type=jax.ShapeDtypeStruct((_NK, _DH), jnp.float32),
        mesh=mesh,
        scratch_types=[
            pltpu.VMEM((_CH,), jnp.int32),
            pltpu.VMEM((_CH, _DH), jnp.float32),
            pltpu.SemaphoreType.DMA,
        ],
        compiler_params=pltpu.CompilerParams(use_tc_tiling_on_sc=False),
    )
    def gk(q_hbm, idx_hbm, out_hbm, idx_v, rows_v, sem):
        w = lax.axis_index("s") * 2 + lax.axis_index("c")
        base = w * per_w
        for cc in range(per_w // _CH):
            off = base + cc * _CH
            pltpu.sync_copy(idx_hbm.at[pl.ds(off, _CH)], idx_v)
            pltpu.async_copy(q_hbm.at[idx_v], rows_v, sem).wait()
            pltpu.sync_copy(rows_v, out_hbm.at[pl.ds(off, _CH)])

    return gk(q, idx_flat)


def _edge_conv(xf, xaT, brow, bcol, wp, wb, b1, g1, be1, w2, b2):
    f = xf.shape[1]
    grid = (_N // _ROWS,)
    idx, p, q = pl.pallas_call(
        _knn_pq_body,
        grid=grid,
        in_specs=[
            pl.BlockSpec((_ROWS, f), lambda i: (i, 0)),
            pl.BlockSpec((f, _N), lambda i: (0, 0)),
            pl.BlockSpec((_ROWS, 1), lambda i: (i, 0)),
            pl.BlockSpec((1, _N), lambda i: (0, 0)),
            pl.BlockSpec((f, _DH), lambda i: (0, 0)),
            pl.BlockSpec((f, _DH), lambda i: (0, 0)),
            pl.BlockSpec((1, _DH), lambda i: (0, 0)),
        ],
        out_specs=[
            pl.BlockSpec((_ROWS, _K), lambda i: (i, 0)),
            pl.BlockSpec((_ROWS, _DH), lambda i: (i, 0)),
            pl.BlockSpec((_ROWS, _DH), lambda i: (i, 0)),
        ],
        out_shape=[
            jax.ShapeDtypeStruct((_N, _K), jnp.int32),
            jax.ShapeDtypeStruct((_N, _DH), jnp.float32),
            jax.ShapeDtypeStruct((_N, _DH), jnp.float32),
        ],
    )(xf, xaT, brow, bcol, wp, wb, b1)

    qg = _gather_rows(q, idx.reshape(_NK))
    qg3 = qg.reshape(_N, _K, _DH)

    acc = pl.pallas_call(
        _stats_body,
        grid=grid,
        in_specs=[
            pl.BlockSpec((_ROWS, _K, _DH), lambda i: (i, 0, 0)),
            pl.BlockSpec((_ROWS, _DH), lambda i: (i, 0)),
        ],
        out_specs=pl.BlockSpec((2, _DH), lambda i: (0, 0)),
        out_shape=jax.ShapeDtypeStruct((2, _DH), jnp.float32),
        compiler_params=pltpu.CompilerParams(
            dimension_semantics=("arbitrary",)),
    )(qg3, p)

    xo = pl.pallas_call(
        _edge_out_body,
        grid=grid,
        in_specs=[
            pl.BlockSpec((_ROWS, _K, _DH), lambda i: (i, 0, 0)),
            pl.BlockSpec((_ROWS, _DH), lambda i: (i, 0)),
            pl.BlockSpec((2, _DH), lambda i: (0, 0)),
            pl.BlockSpec((1, _DH), lambda i: (0, 0)),
            pl.BlockSpec((1, _DH), lambda i: (0, 0)),
            pl.BlockSpec((_DH, _DH), lambda i: (0, 0)),
            pl.BlockSpec((1, _DH), lambda i: (0, 0)),
        ],
        out_specs=pl.BlockSpec((_ROWS, _DH), lambda i: (i, 0)),
        out_shape=jax.ShapeDtypeStruct((_N, _DH), jnp.float32),
    )(qg3, p, acc, g1, be1, w2, b2)
    return xo


def kernel(x, batch, c1_W1, c1_b1, c1_g1, c1_be1, c1_W2, c1_b2,
           c2_W1, c2_b1, c2_g1, c2_be1, c2_W2, c2_b2,
           c3_W1, c3_b1, c3_g1, c3_be1, c3_W2, c3_b2,
           m_W1, m_b1, m_W2, m_b2, m_W3, m_b3, m_W4, m_b4):
    batch = batch.astype(jnp.int32)
    brow = batch.reshape(_N, 1)
    bcol = batch.reshape(1, _N)

    xf1 = jnp.pad(x, ((0, 0), (0, 7)))
    wp1 = jnp.pad(c1_W1[0:1] - c1_W1[1:2], ((0, 7), (0, 0)))
    wb1 = jnp.pad(c1_W1[1:2], ((0, 7), (0, 0)))
    x1 = _edge_conv(xf1, xf1.T, brow, bcol, wp1, wb1,
                    c1_b1.reshape(1, _DH), c1_g1.reshape(1, _DH),
                    c1_be1.reshape(1, _DH), c1_W2, c1_b2.reshape(1, _DH))

    x2 = _edge_conv(x1, x1.T, brow, bcol,
                    c2_W1[:_DH] - c2_W1[_DH:], c2_W1[_DH:],
                    c2_b1.reshape(1, _DH), c2_g1.reshape(1, _DH),
                    c2_be1.reshape(1, _DH), c2_W2, c2_b2.reshape(1, _DH))

    x3 = _edge_conv(x2, x2.T, brow, bcol,
                    c3_W1[:_DH] - c3_W1[_DH:], c3_W1[_DH:],
                    c3_b1.reshape(1, _DH), c3_g1.reshape(1, _DH),
                    c3_be1.reshape(1, _DH), c3_W2, c3_b2.reshape(1, _DH))

    xc = jnp.concatenate([x1, x2, x3], axis=1)
    rb = 1024
    out = pl.pallas_call(
        _mlp_body,
        grid=(_N // rb,),
        in_specs=[pl.BlockSpec((rb, 192), lambda i: (i, 0))] + [
            pl.BlockSpec(s, lambda i: (0, 0)) for s in
            [(192, 256), (1, 256), (256, 128), (1, 128),
             (128, 64), (1, 64), (64, 16), (1, 16)]
        ],
        out_specs=pl.BlockSpec((rb, 16), lambda i: (i, 0)),
        out_shape=jax.ShapeDtypeStruct((_N, 16), jnp.float32),
    )(xc, m_W1, m_b1.reshape(1, 256), m_W2, m_b2.reshape(1, 128),
      m_W3, m_b3.reshape(1, 64), m_W4, m_b4.reshape(1, 16))
    return out


# speedup vs baseline: 6.1002x; 1.1158x over previous
import functools

import jax
import jax.numpy as jnp
from jax import lax
from jax.experimental import pallas as pl
from jax.experimental.pallas import tpu as pltpu
from jax.experimental.pallas import tpu_sc as plsc

_N = 4096
_K = 16
_NK = _N * _K
_EPS = 1e-5
_DH = 64
_ROWS = 256
_W = 1536
_W2 = 2560
_NW = 32
_CH = 128


def _topk_from_key(key, r):
    kiota = lax.broadcasted_iota(jnp.int32, (r, _K), 1)
    idx_acc = jnp.zeros((r, _K), jnp.int32)
    for k in range(_K):
        m = jnp.min(key, axis=1, keepdims=True)
        key = jnp.where(key == m, jnp.int32(0x7FFFFFFF), key)
        idx_acc = jnp.where(kiota == k, m & 0xFFF, idx_acc)
    return idx_acc


def _knn_pq_body(xf_ref, xaT_ref, brow_ref, bcol_ref, wp_ref, wb_ref, b1_ref,
                 idx_ref, p_ref, q_ref):
    xb = xf_ref[...]
    r = xb.shape[0]
    sqb = jnp.sum(xb * xb, axis=1, keepdims=True)
    brow = brow_ref[...]

    def compute_key(xaT, bcol, base):
        prod = lax.dot_general(xb, xaT, (((1,), (0,)), ((), ())),
                               preferred_element_type=jnp.float32)
        sqa = jnp.sum(xaT * xaT, axis=0, keepdims=True)
        d = (sqb + sqa) - 2.0 * prod
        d = jnp.where(brow != bcol, jnp.inf, d)
        d = jnp.maximum(d, 0.0)
        iota = lax.broadcasted_iota(jnp.int32, d.shape, 1) + base
        return (lax.bitcast_convert_type(d, jnp.int32)
                & jnp.int32(~0xFFF)) | iota

    b_first = brow_ref[0, 0]
    b_last = brow_ref[r - 1, 0]
    bcol_full = bcol_ref[...]
    citer = lax.broadcasted_iota(jnp.int32, (1, _N), 1)
    lo = jnp.min(jnp.where(bcol_full == b_first, citer, _N))
    hi = jnp.max(jnp.where(bcol_full == b_last, citer, -1)) + 1
    lo128 = (lo // 128) * 128
    lo_a = jnp.minimum(lo128, _N - _W)
    lo_b = jnp.minimum(lo128, _N - _W2)

    def wpath(_):
        return _topk_from_key(
            compute_key(xaT_ref[:, pl.ds(lo_a, _W)],
                        bcol_ref[:, pl.ds(lo_a, _W)], lo_a), r)

    def mpath(_):
        return _topk_from_key(
            compute_key(xaT_ref[:, pl.ds(lo_b, _W2)],
                        bcol_ref[:, pl.ds(lo_b, _W2)], lo_b), r)

    def fpath(_):
        return _topk_from_key(compute_key(xaT_ref[...], bcol_full, 0), r)

    minsz = jnp.int32(_N)
    for g in range(4):
        sz = jnp.sum(jnp.where(bcol_full == g, 1, 0))
        ing = (g >= b_first) & (g <= b_last)
        minsz = jnp.where(ing, jnp.minimum(minsz, sz), minsz)
    ok = minsz >= _K

    idx_ref[...] = lax.cond(
        ((hi - lo_a) <= _W) & ok, wpath,
        lambda u: lax.cond(((hi - lo_b) <= _W2) & ok, mpath, fpath, u), 0)
    p_ref[...] = xb @ wp_ref[...] + b1_ref[...]
    q_ref[...] = xb @ wb_ref[...]


def _stats_body(qg_ref, p_ref, acc_ref):
    qg = qg_ref[...].reshape(_ROWS, _K, _DH)
    h = p_ref[...][:, None, :] + qg
    s = jnp.sum(jnp.sum(h, axis=1), axis=0, keepdims=True)
    s2 = jnp.sum(jnp.sum(h * h, axis=1), axis=0, keepdims=True)
    val = jnp.concatenate([s, s2], axis=0)
    @pl.when(pl.program_id(0) == 0)
    def _():
        acc_ref[...] = val
    @pl.when(pl.program_id(0) != 0)
    def _():
        acc_ref[...] = acc_ref[...] + val


def _edge_out_body(qg_ref, p_ref, acc_ref, g_ref, be_ref, w2_ref, b2_ref,
                   o_ref):
    inv_n = 1.0 / float(_NK)
    s = acc_ref[0:1, :]
    s2 = acc_ref[1:2, :]
    mean = s * inv_n
    var = s2 * inv_n - mean * mean
    a = g_ref[...] * lax.rsqrt(var + _EPS)
    c = be_ref[...] - a * mean
    qg = qg_ref[...].reshape(_ROWS, _K, _DH)
    h = p_ref[...][:, None, :] + qg
    hn = jnp.maximum(h * a[:, None, :] + c[:, None, :], 0.0)
    y = hn.reshape(_ROWS * _K, _DH) @ w2_ref[...] + b2_ref[...]
    o_ref[...] = jnp.max(y.reshape(_ROWS, _K, _DH), axis=1)


def _mlp_body(xc_ref, w1_ref, b1_ref, w2_ref, b2_ref, w3_ref, b3_ref,
              w4_ref, b4_ref, o_ref):
    h = xc_ref[...]
    h = jnp.maximum(h @ w1_ref[...] + b1_ref[...], 0.0)
    h = jnp.maximum(h @ w2_ref[...] + b2_ref[...], 0.0)
    h = jnp.maximum(h @ w3_ref[...] + b3_ref[...], 0.0)
    h = h @ w4_ref[...] + b4_ref[...]
    z = h - jnp.max(h, axis=1, keepdims=True)
    o_ref[...] = z - jnp.log(jnp.sum(jnp.exp(z), axis=1, keepdims=True))


def _gather_rows(q, idx_flat):
    per_w = _NK // _NW
    mesh = plsc.VectorSubcoreMesh(core_axis_name="c", subcore_axis_name="s")

    n_ch = per_w // _CH

    @functools.partial(
        pl.kernel,
        out_type=jax.ShapeDtypeStruct((_NK, _DH), jnp.float32),
        mesh=mesh,
        scratch_types=[
            pltpu.VMEM((per_w,), jnp.int32),
            pltpu.VMEM((_CH, _DH), jnp.float32),
            pltpu.VMEM((_CH, _DH), jnp.float32),
            pltpu.SemaphoreType.DMA,
            pltpu.SemaphoreType.DMA,
        ],
        compiler_params=pltpu.CompilerParams(use_tc_tiling_on_sc=False),
    )
    def gk(q_hbm, idx_hbm, out_hbm, idx_v, buf0, buf1, sem0, sem1):
        w = lax.axis_index("s") * 2 + lax.axis_index("c")
        base = w * per_w
        pltpu.sync_copy(idx_hbm.at[pl.ds(base, per_w)], idx_v)
        bufs = (buf0, buf1)
        sems = (sem0, sem1)
        copies = [None] * n_ch
        copies[0] = pltpu.async_copy(
            q_hbm.at[idx_v.at[pl.ds(0, _CH)]], bufs[0], sems[0])
        for cc in range(n_ch):
            if cc + 1 < n_ch:
                copies[cc + 1] = pltpu.async_copy(
                    q_hbm.at[idx_v.at[pl.ds((cc + 1) * _CH, _CH)]],
                    bufs[(cc + 1) % 2], sems[(cc + 1) % 2])
            copies[cc].wait()
            pltpu.sync_copy(bufs[cc % 2],
                            out_hbm.at[pl.ds(base + cc * _CH, _CH)])

    return gk(q, idx_flat)


def _edge_conv(xf, xaT, brow, bcol, wp, wb, b1, g1, be1, w2, b2):
    f = xf.shape[1]
    grid = (_N // _ROWS,)
    idx, p, q = pl.pallas_call(
        _knn_pq_body,
        grid=grid,
        in_specs=[
            pl.BlockSpec((_ROWS, f), lambda i: (i, 0)),
            pl.BlockSpec((f, _N), lambda i: (0, 0)),
            pl.BlockSpec((_ROWS, 1), lambda i: (i, 0)),
            pl.BlockSpec((1, _N), lambda i: (0, 0)),
            pl.BlockSpec((f, _DH), lambda i: (0, 0)),
            pl.BlockSpec((f, _DH), lambda i: (0, 0)),
            pl.BlockSpec((1, _DH), lambda i: (0, 0)),
        ],
        out_specs=[
            pl.BlockSpec((_ROWS, _K), lambda i: (i, 0)),
            pl.BlockSpec((_ROWS, _DH), lambda i: (i, 0)),
            pl.BlockSpec((_ROWS, _DH), lambda i: (i, 0)),
        ],
        out_shape=[
            jax.ShapeDtypeStruct((_N, _K), jnp.int32),
            jax.ShapeDtypeStruct((_N, _DH), jnp.float32),
            jax.ShapeDtypeStruct((_N, _DH), jnp.float32),
        ],
    )(xf, xaT, brow, bcol, wp, wb, b1)

    qg = _gather_rows(q, idx.reshape(_NK))

    acc = pl.pallas_call(
        _stats_body,
        grid=grid,
        in_specs=[
            pl.BlockSpec((_ROWS * _K, _DH), lambda i: (i, 0)),
            pl.BlockSpec((_ROWS, _DH), lambda i: (i, 0)),
        ],
        out_specs=pl.BlockSpec((2, _DH), lambda i: (0, 0)),
        out_shape=jax.ShapeDtypeStruct((2, _DH), jnp.float32),
        compiler_params=pltpu.CompilerParams(
            dimension_semantics=("arbitrary",)),
    )(qg, p)

    xo = pl.pallas_call(
        _edge_out_body,
        grid=grid,
        in_specs=[
            pl.BlockSpec((_ROWS * _K, _DH), lambda i: (i, 0)),
            pl.BlockSpec((_ROWS, _DH), lambda i: (i, 0)),
            pl.BlockSpec((2, _DH), lambda i: (0, 0)),
            pl.BlockSpec((1, _DH), lambda i: (0, 0)),
            pl.BlockSpec((1, _DH), lambda i: (0, 0)),
            pl.BlockSpec((_DH, _DH), lambda i: (0, 0)),
            pl.BlockSpec((1, _DH), lambda i: (0, 0)),
        ],
        out_specs=pl.BlockSpec((_ROWS, _DH), lambda i: (i, 0)),
        out_shape=jax.ShapeDtypeStruct((_N, _DH), jnp.float32),
    )(qg, p, acc, g1, be1, w2, b2)
    return xo


def kernel(x, batch, c1_W1, c1_b1, c1_g1, c1_be1, c1_W2, c1_b2,
           c2_W1, c2_b1, c2_g1, c2_be1, c2_W2, c2_b2,
           c3_W1, c3_b1, c3_g1, c3_be1, c3_W2, c3_b2,
           m_W1, m_b1, m_W2, m_b2, m_W3, m_b3, m_W4, m_b4):
    batch = batch.astype(jnp.int32)
    brow = batch.reshape(_N, 1)
    bcol = batch.reshape(1, _N)

    xf1 = jnp.pad(x, ((0, 0), (0, 7)))
    wp1 = jnp.pad(c1_W1[0:1] - c1_W1[1:2], ((0, 7), (0, 0)))
    wb1 = jnp.pad(c1_W1[1:2], ((0, 7), (0, 0)))
    x1 = _edge_conv(xf1, xf1.T, brow, bcol, wp1, wb1,
                    c1_b1.reshape(1, _DH), c1_g1.reshape(1, _DH),
                    c1_be1.reshape(1, _DH), c1_W2, c1_b2.reshape(1, _DH))

    x2 = _edge_conv(x1, x1.T, brow, bcol,
                    c2_W1[:_DH] - c2_W1[_DH:], c2_W1[_DH:],
                    c2_b1.reshape(1, _DH), c2_g1.reshape(1, _DH),
                    c2_be1.reshape(1, _DH), c2_W2, c2_b2.reshape(1, _DH))

    x3 = _edge_conv(x2, x2.T, brow, bcol,
                    c3_W1[:_DH] - c3_W1[_DH:], c3_W1[_DH:],
                    c3_b1.reshape(1, _DH), c3_g1.reshape(1, _DH),
                    c3_be1.reshape(1, _DH), c3_W2, c3_b2.reshape(1, _DH))

    xc = jnp.concatenate([x1, x2, x3], axis=1)
    rb = 1024
    out = pl.pallas_call(
        _mlp_body,
        grid=(_N // rb,),
        in_specs=[pl.BlockSpec((rb, 192), lambda i: (i, 0))] + [
            pl.BlockSpec(s, lambda i: (0, 0)) for s in
            [(192, 256), (1, 256), (256, 128), (1, 128),
             (128, 64), (1, 64), (64, 16), (1, 16)]
        ],
        out_specs=pl.BlockSpec((rb, 16), lambda i: (i, 0)),
        out_shape=jax.ShapeDtypeStruct((_N, 16), jnp.float32),
    )(xc, m_W1, m_b1.reshape(1, 256), m_W2, m_b2.reshape(1, 128),
      m_W3, m_b3.reshape(1, 64), m_W4, m_b4.reshape(1, 16))
    return out


# speedup vs baseline: 7.3703x; 1.2082x over previous
import functools

import jax
import jax.numpy as jnp
from jax import lax
from jax.experimental import pallas as pl
from jax.experimental.pallas import tpu as pltpu
from jax.experimental.pallas import tpu_sc as plsc

_N = 4096
_K = 16
_NK = _N * _K
_EPS = 1e-5
_DH = 64
_ROWS = 256
_W = 1536
_W2 = 2560
_NW = 32
_CH = 128


_FBIG = 1.7014118346046923e38
_FBIG2 = 1.70141e38
_FMAX = 3.4028234663852886e38


def _topk_from_key(key_i, r):
    key = lax.bitcast_convert_type(key_i, jnp.float32)
    kiota = lax.broadcasted_iota(jnp.int32, (r, _K), 1)
    idx_acc = jnp.zeros((r, _K), jnp.int32)
    m = jnp.min(key, axis=1, keepdims=True)
    for k in range(_K):
        mi = lax.bitcast_convert_type(m, jnp.int32) & 0xFFF
        idx_acc = jnp.where(kiota == k, mi, idx_acc)
        if k + 1 < _K:
            m = jnp.min(jnp.where(key > m, key, _FMAX), axis=1,
                        keepdims=True)
    return idx_acc


def _knn_pq_body(xf_ref, xaT_ref, brow_ref, bcol_ref, wp_ref, wb_ref, b1_ref,
                 idx_ref, p_ref, q_ref):
    xb = xf_ref[...]
    r = xb.shape[0]
    sqb = jnp.sum(xb * xb, axis=1, keepdims=True)
    brow = brow_ref[...]

    def compute_key(xaT, bcol, base):
        prod = lax.dot_general(xb, xaT, (((1,), (0,)), ((), ())),
                               preferred_element_type=jnp.float32)
        sqa = jnp.sum(xaT * xaT, axis=0, keepdims=True)
        d = (sqb + sqa) - 2.0 * prod
        d = jnp.minimum(jnp.maximum(d, 1.1754944e-38), _FBIG2)
        d = jnp.where(brow != bcol, _FBIG, d)
        iota = lax.broadcasted_iota(jnp.int32, d.shape, 1) + base
        return (lax.bitcast_convert_type(d, jnp.int32)
                & jnp.int32(~0xFFF)) | iota

    b_first = brow_ref[0, 0]
    b_last = brow_ref[r - 1, 0]
    bcol_full = bcol_ref[...]
    citer = lax.broadcasted_iota(jnp.int32, (1, _N), 1)
    lo = jnp.min(jnp.where(bcol_full == b_first, citer, _N))
    hi = jnp.max(jnp.where(bcol_full == b_last, citer, -1)) + 1
    lo128 = (lo // 128) * 128
    lo_a = jnp.minimum(lo128, _N - _W)
    lo_b = jnp.minimum(lo128, _N - _W2)

    def wpath(_):
        return _topk_from_key(
            compute_key(xaT_ref[:, pl.ds(lo_a, _W)],
                        bcol_ref[:, pl.ds(lo_a, _W)], lo_a), r)

    def mpath(_):
        return _topk_from_key(
            compute_key(xaT_ref[:, pl.ds(lo_b, _W2)],
                        bcol_ref[:, pl.ds(lo_b, _W2)], lo_b), r)

    def fpath(_):
        return _topk_from_key(compute_key(xaT_ref[...], bcol_full, 0), r)

    minsz = jnp.int32(_N)
    for g in range(4):
        sz = jnp.sum(jnp.where(bcol_full == g, 1, 0))
        ing = (g >= b_first) & (g <= b_last)
        minsz = jnp.where(ing, jnp.minimum(minsz, sz), minsz)
    ok = minsz >= _K

    idx_ref[...] = lax.cond(
        ((hi - lo_a) <= _W) & ok, wpath,
        lambda u: lax.cond(((hi - lo_b) <= _W2) & ok, mpath, fpath, u), 0)
    p_ref[...] = xb @ wp_ref[...] + b1_ref[...]
    q_ref[...] = xb @ wb_ref[...]


def _stats_body(qg_ref, p_ref, acc_ref):
    qg = qg_ref[...].reshape(_ROWS, _K, _DH)
    h = p_ref[...][:, None, :] + qg
    s = jnp.sum(jnp.sum(h, axis=1), axis=0, keepdims=True)
    s2 = jnp.sum(jnp.sum(h * h, axis=1), axis=0, keepdims=True)
    val = jnp.concatenate([s, s2], axis=0)
    @pl.when(pl.program_id(0) == 0)
    def _():
        acc_ref[...] = val
    @pl.when(pl.program_id(0) != 0)
    def _():
        acc_ref[...] = acc_ref[...] + val


def _edge_out_body(qg_ref, p_ref, acc_ref, g_ref, be_ref, w2_ref, b2_ref,
                   o_ref):
    inv_n = 1.0 / float(_NK)
    s = acc_ref[0:1, :]
    s2 = acc_ref[1:2, :]
    mean = s * inv_n
    var = s2 * inv_n - mean * mean
    a = g_ref[...] * lax.rsqrt(var + _EPS)
    c = be_ref[...] - a * mean
    qg = qg_ref[...].reshape(_ROWS, _K, _DH)
    h = p_ref[...][:, None, :] + qg
    hn = jnp.maximum(h * a[:, None, :] + c[:, None, :], 0.0)
    y = hn.reshape(_ROWS * _K, _DH) @ w2_ref[...] + b2_ref[...]
    o_ref[...] = jnp.max(y.reshape(_ROWS, _K, _DH), axis=1)


def _mlp_body(xc_ref, w1_ref, b1_ref, w2_ref, b2_ref, w3_ref, b3_ref,
              w4_ref, b4_ref, o_ref):
    h = xc_ref[...]
    h = jnp.maximum(h @ w1_ref[...] + b1_ref[...], 0.0)
    h = jnp.maximum(h @ w2_ref[...] + b2_ref[...], 0.0)
    h = jnp.maximum(h @ w3_ref[...] + b3_ref[...], 0.0)
    h = h @ w4_ref[...] + b4_ref[...]
    z = h - jnp.max(h, axis=1, keepdims=True)
    o_ref[...] = z - jnp.log(jnp.sum(jnp.exp(z), axis=1, keepdims=True))


def _gather_rows(q, idx_flat):
    per_w = _NK // _NW
    mesh = plsc.VectorSubcoreMesh(core_axis_name="c", subcore_axis_name="s")

    n_ch = per_w // _CH

    @functools.partial(
        pl.kernel,
        out_type=jax.ShapeDtypeStruct((_NK, _DH), jnp.float32),
        mesh=mesh,
        scratch_types=[
            pltpu.VMEM((per_w,), jnp.int32),
            pltpu.VMEM((_CH, _DH), jnp.float32),
            pltpu.VMEM((_CH, _DH), jnp.float32),
            pltpu.SemaphoreType.DMA,
            pltpu.SemaphoreType.DMA,
        ],
        compiler_params=pltpu.CompilerParams(use_tc_tiling_on_sc=False),
    )
    def gk(q_hbm, idx_hbm, out_hbm, idx_v, buf0, buf1, sem0, sem1):
        w = lax.axis_index("s") * 2 + lax.axis_index("c")
        base = w * per_w
        pltpu.sync_copy(idx_hbm.at[pl.ds(base, per_w)], idx_v)
        bufs = (buf0, buf1)
        sems = (sem0, sem1)
        copies = [None] * n_ch
        copies[0] = pltpu.async_copy(
            q_hbm.at[idx_v.at[pl.ds(0, _CH)]], bufs[0], sems[0])
        for cc in range(n_ch):
            if cc + 1 < n_ch:
                copies[cc + 1] = pltpu.async_copy(
                    q_hbm.at[idx_v.at[pl.ds((cc + 1) * _CH, _CH)]],
                    bufs[(cc + 1) % 2], sems[(cc + 1) % 2])
            copies[cc].wait()
            pltpu.sync_copy(bufs[cc % 2],
                            out_hbm.at[pl.ds(base + cc * _CH, _CH)])

    return gk(q, idx_flat)


def _edge_conv(xf, xaT, brow, bcol, wp, wb, b1, g1, be1, w2, b2):
    f = xf.shape[1]
    grid = (_N // _ROWS,)
    idx, p, q = pl.pallas_call(
        _knn_pq_body,
        grid=grid,
        in_specs=[
            pl.BlockSpec((_ROWS, f), lambda i: (i, 0)),
            pl.BlockSpec((f, _N), lambda i: (0, 0)),
            pl.BlockSpec((_ROWS, 1), lambda i: (i, 0)),
            pl.BlockSpec((1, _N), lambda i: (0, 0)),
            pl.BlockSpec((f, _DH), lambda i: (0, 0)),
            pl.BlockSpec((f, _DH), lambda i: (0, 0)),
            pl.BlockSpec((1, _DH), lambda i: (0, 0)),
        ],
        out_specs=[
            pl.BlockSpec((_ROWS, _K), lambda i: (i, 0)),
            pl.BlockSpec((_ROWS, _DH), lambda i: (i, 0)),
            pl.BlockSpec((_ROWS, _DH), lambda i: (i, 0)),
        ],
        out_shape=[
            jax.ShapeDtypeStruct((_N, _K), jnp.int32),
            jax.ShapeDtypeStruct((_N, _DH), jnp.float32),
            jax.ShapeDtypeStruct((_N, _DH), jnp.float32),
        ],
    )(xf, xaT, brow, bcol, wp, wb, b1)

    qg = _gather_rows(q, idx.reshape(_NK))

    acc = pl.pallas_call(
        _stats_body,
        grid=grid,
        in_specs=[
            pl.BlockSpec((_ROWS * _K, _DH), lambda i: (i, 0)),
            pl.BlockSpec((_ROWS, _DH), lambda i: (i, 0)),
        ],
        out_specs=pl.BlockSpec((2, _DH), lambda i: (0, 0)),
        out_shape=jax.ShapeDtypeStruct((2, _DH), jnp.float32),
        compiler_params=pltpu.CompilerParams(
            dimension_semantics=("arbitrary",)),
    )(qg, p)

    xo = pl.pallas_call(
        _edge_out_body,
        grid=grid,
        in_specs=[
            pl.BlockSpec((_ROWS * _K, _DH), lambda i: (i, 0)),
            pl.BlockSpec((_ROWS, _DH), lambda i: (i, 0)),
            pl.BlockSpec((2, _DH), lambda i: (0, 0)),
            pl.BlockSpec((1, _DH), lambda i: (0, 0)),
            pl.BlockSpec((1, _DH), lambda i: (0, 0)),
            pl.BlockSpec((_DH, _DH), lambda i: (0, 0)),
            pl.BlockSpec((1, _DH), lambda i: (0, 0)),
        ],
        out_specs=pl.BlockSpec((_ROWS, _DH), lambda i: (i, 0)),
        out_shape=jax.ShapeDtypeStruct((_N, _DH), jnp.float32),
    )(qg, p, acc, g1, be1, w2, b2)
    return xo


def kernel(x, batch, c1_W1, c1_b1, c1_g1, c1_be1, c1_W2, c1_b2,
           c2_W1, c2_b1, c2_g1, c2_be1, c2_W2, c2_b2,
           c3_W1, c3_b1, c3_g1, c3_be1, c3_W2, c3_b2,
           m_W1, m_b1, m_W2, m_b2, m_W3, m_b3, m_W4, m_b4):
    batch = batch.astype(jnp.int32)
    brow = batch.reshape(_N, 1)
    bcol = batch.reshape(1, _N)

    xf1 = jnp.pad(x, ((0, 0), (0, 7)))
    wp1 = jnp.pad(c1_W1[0:1] - c1_W1[1:2], ((0, 7), (0, 0)))
    wb1 = jnp.pad(c1_W1[1:2], ((0, 7), (0, 0)))
    x1 = _edge_conv(xf1, xf1.T, brow, bcol, wp1, wb1,
                    c1_b1.reshape(1, _DH), c1_g1.reshape(1, _DH),
                    c1_be1.reshape(1, _DH), c1_W2, c1_b2.reshape(1, _DH))

    x2 = _edge_conv(x1, x1.T, brow, bcol,
                    c2_W1[:_DH] - c2_W1[_DH:], c2_W1[_DH:],
                    c2_b1.reshape(1, _DH), c2_g1.reshape(1, _DH),
                    c2_be1.reshape(1, _DH), c2_W2, c2_b2.reshape(1, _DH))

    x3 = _edge_conv(x2, x2.T, brow, bcol,
                    c3_W1[:_DH] - c3_W1[_DH:], c3_W1[_DH:],
                    c3_b1.reshape(1, _DH), c3_g1.reshape(1, _DH),
                    c3_be1.reshape(1, _DH), c3_W2, c3_b2.reshape(1, _DH))

    xc = jnp.concatenate([x1, x2, x3], axis=1)
    rb = 1024
    out = pl.pallas_call(
        _mlp_body,
        grid=(_N // rb,),
        in_specs=[pl.BlockSpec((rb, 192), lambda i: (i, 0))] + [
            pl.BlockSpec(s, lambda i: (0, 0)) for s in
            [(192, 256), (1, 256), (256, 128), (1, 128),
             (128, 64), (1, 64), (64, 16), (1, 16)]
        ],
        out_specs=pl.BlockSpec((rb, 16), lambda i: (i, 0)),
        out_shape=jax.ShapeDtypeStruct((_N, 16), jnp.float32),
    )(xc, m_W1, m_b1.reshape(1, 256), m_W2, m_b2.reshape(1, 128),
      m_W3, m_b3.reshape(1, 64), m_W4, m_b4.reshape(1, 16))
    return out


# speedup vs baseline: 8.1060x; 1.0998x over previous
import functools

import jax
import jax.numpy as jnp
from jax import lax
from jax.experimental import pallas as pl
from jax.experimental.pallas import tpu as pltpu
from jax.experimental.pallas import tpu_sc as plsc

_N = 4096
_K = 16
_NK = _N * _K
_EPS = 1e-5
_DH = 64
_ROWS = 256
_W = 1536
_W2 = 2560
_NW = 32
_CH = 128


_FBIG = 1.7014118346046923e38
_FBIG2 = 1.70141e38
_FMAX = 3.4028234663852886e38


def _topk_from_key(key_i, r):
    key = lax.bitcast_convert_type(key_i, jnp.float32)
    kiota = lax.broadcasted_iota(jnp.int32, (r, _K), 1)
    idx_acc = jnp.zeros((r, _K), jnp.int32)
    m = jnp.min(key, axis=1, keepdims=True)
    for k in range(_K):
        mi = lax.bitcast_convert_type(m, jnp.int32) & 0xFFF
        idx_acc = jnp.where(kiota == k, mi, idx_acc)
        if k + 1 < _K:
            m = jnp.min(jnp.where(key > m, key, _FMAX), axis=1,
                        keepdims=True)
    return idx_acc


def _knn_pq_body(xf_ref, xaT_ref, brow_ref, bcol_ref, wp_ref, wb_ref, b1_ref,
                 idx_ref, p_ref, q_ref):
    xb = xf_ref[...]
    r = xb.shape[0]
    sqb = jnp.sum(xb * xb, axis=1, keepdims=True)
    brow = brow_ref[...]

    def compute_key(xaT, bcol, base):
        prod = lax.dot_general(xb, xaT, (((1,), (0,)), ((), ())),
                               preferred_element_type=jnp.float32)
        sqa = jnp.sum(xaT * xaT, axis=0, keepdims=True)
        d = (sqb + sqa) - 2.0 * prod
        d = jnp.minimum(jnp.maximum(d, 1.1754944e-38), _FBIG2)
        d = jnp.where(brow != bcol, _FBIG, d)
        iota = lax.broadcasted_iota(jnp.int32, d.shape, 1) + base
        return (lax.bitcast_convert_type(d, jnp.int32)
                & jnp.int32(~0xFFF)) | iota

    b_first = brow_ref[0, 0]
    b_last = brow_ref[r - 1, 0]
    bcol_full = bcol_ref[...]
    citer = lax.broadcasted_iota(jnp.int32, (1, _N), 1)
    lo = jnp.min(jnp.where(bcol_full == b_first, citer, _N))
    hi = jnp.max(jnp.where(bcol_full == b_last, citer, -1)) + 1
    lo128 = (lo // 128) * 128
    lo_a = jnp.minimum(lo128, _N - _W)
    lo_b = jnp.minimum(lo128, _N - _W2)

    def wpath(_):
        return _topk_from_key(
            compute_key(xaT_ref[:, pl.ds(lo_a, _W)],
                        bcol_ref[:, pl.ds(lo_a, _W)], lo_a), r)

    def mpath(_):
        return _topk_from_key(
            compute_key(xaT_ref[:, pl.ds(lo_b, _W2)],
                        bcol_ref[:, pl.ds(lo_b, _W2)], lo_b), r)

    def fpath(_):
        return _topk_from_key(compute_key(xaT_ref[...], bcol_full, 0), r)

    minsz = jnp.int32(_N)
    for g in range(4):
        sz = jnp.sum(jnp.where(bcol_full == g, 1, 0))
        ing = (g >= b_first) & (g <= b_last)
        minsz = jnp.where(ing, jnp.minimum(minsz, sz), minsz)
    ok = minsz >= _K

    idx_ref[...] = lax.cond(
        ((hi - lo_a) <= _W) & ok, wpath,
        lambda u: lax.cond(((hi - lo_b) <= _W2) & ok, mpath, fpath, u), 0)
    p_ref[...] = xb @ wp_ref[...] + b1_ref[...]
    q_ref[...] = xb @ wb_ref[...]


def _stats_body(qg_ref, p_ref, acc_ref):
    p = p_ref[...]
    s = jnp.zeros((1, _DH), jnp.float32)
    s2 = jnp.zeros((1, _DH), jnp.float32)
    for k in range(_K):
        h = qg_ref[k, :, :_DH] + p
        s = s + jnp.sum(h, axis=0, keepdims=True)
        s2 = s2 + jnp.sum(h * h, axis=0, keepdims=True)
    val = jnp.concatenate([s, s2], axis=0)
    @pl.when(pl.program_id(0) == 0)
    def _():
        acc_ref[...] = val
    @pl.when(pl.program_id(0) != 0)
    def _():
        acc_ref[...] = acc_ref[...] + val


def _edge_out_body(qg_ref, p_ref, acc_ref, g_ref, be_ref, w2_ref, b2_ref,
                   o_ref):
    inv_n = 1.0 / float(_NK)
    s = acc_ref[0:1, :]
    s2 = acc_ref[1:2, :]
    mean = s * inv_n
    var = s2 * inv_n - mean * mean
    a = g_ref[...] * lax.rsqrt(var + _EPS)
    c = be_ref[...] - a * mean
    p = p_ref[...]
    w2 = w2_ref[...]
    b2 = b2_ref[...]
    acc = jnp.full((_ROWS, _DH), -_FMAX, jnp.float32)
    for k in range(_K):
        h = qg_ref[k, :, :_DH] + p
        hn = jnp.maximum(h * a + c, 0.0)
        acc = jnp.maximum(acc, hn @ w2 + b2)
    o_ref[...] = acc


def _mlp_body(xc_ref, w1_ref, b1_ref, w2_ref, b2_ref, w3_ref, b3_ref,
              w4_ref, b4_ref, o_ref):
    h = xc_ref[...]
    h = jnp.maximum(h @ w1_ref[...] + b1_ref[...], 0.0)
    h = jnp.maximum(h @ w2_ref[...] + b2_ref[...], 0.0)
    h = jnp.maximum(h @ w3_ref[...] + b3_ref[...], 0.0)
    h = h @ w4_ref[...] + b4_ref[...]
    z = h - jnp.max(h, axis=1, keepdims=True)
    o_ref[...] = z - jnp.log(jnp.sum(jnp.exp(z), axis=1, keepdims=True))


def _gather_rows(q, idx_flat):
    per_w = _NK // _NW
    mesh = plsc.VectorSubcoreMesh(core_axis_name="c", subcore_axis_name="s")

    n_ch = per_w // _CH

    @functools.partial(
        pl.kernel,
        out_type=jax.ShapeDtypeStruct((_NK, 2 * _DH), jnp.float32),
        mesh=mesh,
        scratch_types=[
            pltpu.VMEM((per_w,), jnp.int32),
            pltpu.VMEM((_CH, 2 * _DH), jnp.float32),
            pltpu.VMEM((_CH, 2 * _DH), jnp.float32),
            pltpu.SemaphoreType.DMA,
            pltpu.SemaphoreType.DMA,
        ],
    )
    def gk(q_hbm, idx_hbm, out_hbm, idx_v, buf0, buf1, sem0, sem1):
        w = lax.axis_index("s") * 2 + lax.axis_index("c")
        base = w * per_w
        pltpu.sync_copy(idx_hbm.at[pl.ds(base, per_w)], idx_v)
        bufs = (buf0, buf1)
        sems = (sem0, sem1)
        copies = [None] * n_ch
        copies[0] = pltpu.async_copy(
            q_hbm.at[idx_v.at[pl.ds(0, _CH)]], bufs[0], sems[0])
        for cc in range(n_ch):
            if cc + 1 < n_ch:
                copies[cc + 1] = pltpu.async_copy(
                    q_hbm.at[idx_v.at[pl.ds((cc + 1) * _CH, _CH)]],
                    bufs[(cc + 1) % 2], sems[(cc + 1) % 2])
            copies[cc].wait()
            pltpu.sync_copy(bufs[cc % 2],
                            out_hbm.at[pl.ds(base + cc * _CH, _CH)])

    return gk(q, idx_flat)


def _edge_conv(xf, xaT, brow, bcol, wp, wb, b1, g1, be1, w2, b2):
    f = xf.shape[1]
    wb = jnp.pad(wb, ((0, 0), (0, _DH)))
    grid = (_N // _ROWS,)
    idx, p, q = pl.pallas_call(
        _knn_pq_body,
        grid=grid,
        in_specs=[
            pl.BlockSpec((_ROWS, f), lambda i: (i, 0)),
            pl.BlockSpec((f, _N), lambda i: (0, 0)),
            pl.BlockSpec((_ROWS, 1), lambda i: (i, 0)),
            pl.BlockSpec((1, _N), lambda i: (0, 0)),
            pl.BlockSpec((f, _DH), lambda i: (0, 0)),
            pl.BlockSpec((f, 2 * _DH), lambda i: (0, 0)),
            pl.BlockSpec((1, _DH), lambda i: (0, 0)),
        ],
        out_specs=[
            pl.BlockSpec((_ROWS, _K), lambda i: (i, 0)),
            pl.BlockSpec((_ROWS, _DH), lambda i: (i, 0)),
            pl.BlockSpec((_ROWS, 2 * _DH), lambda i: (i, 0)),
        ],
        out_shape=[
            jax.ShapeDtypeStruct((_N, _K), jnp.int32),
            jax.ShapeDtypeStruct((_N, _DH), jnp.float32),
            jax.ShapeDtypeStruct((_N, 2 * _DH), jnp.float32),
        ],
    )(xf, xaT, brow, bcol, wp, wb, b1)

    qg = _gather_rows(q, idx.T.reshape(_NK))
    qg3 = qg.reshape(_K, _N, 2 * _DH)

    acc = pl.pallas_call(
        _stats_body,
        grid=grid,
        in_specs=[
            pl.BlockSpec((_K, _ROWS, 2 * _DH), lambda i: (0, i, 0)),
            pl.BlockSpec((_ROWS, _DH), lambda i: (i, 0)),
        ],
        out_specs=pl.BlockSpec((2, _DH), lambda i: (0, 0)),
        out_shape=jax.ShapeDtypeStruct((2, _DH), jnp.float32),
        compiler_params=pltpu.CompilerParams(
            dimension_semantics=("arbitrary",)),
    )(qg3, p)

    xo = pl.pallas_call(
        _edge_out_body,
        grid=grid,
        in_specs=[
            pl.BlockSpec((_K, _ROWS, 2 * _DH), lambda i: (0, i, 0)),
            pl.BlockSpec((_ROWS, _DH), lambda i: (i, 0)),
            pl.BlockSpec((2, _DH), lambda i: (0, 0)),
            pl.BlockSpec((1, _DH), lambda i: (0, 0)),
            pl.BlockSpec((1, _DH), lambda i: (0, 0)),
            pl.BlockSpec((_DH, _DH), lambda i: (0, 0)),
            pl.BlockSpec((1, _DH), lambda i: (0, 0)),
        ],
        out_specs=pl.BlockSpec((_ROWS, _DH), lambda i: (i, 0)),
        out_shape=jax.ShapeDtypeStruct((_N, _DH), jnp.float32),
    )(qg3, p, acc, g1, be1, w2, b2)
    return xo


def kernel(x, batch, c1_W1, c1_b1, c1_g1, c1_be1, c1_W2, c1_b2,
           c2_W1, c2_b1, c2_g1, c2_be1, c2_W2, c2_b2,
           c3_W1, c3_b1, c3_g1, c3_be1, c3_W2, c3_b2,
           m_W1, m_b1, m_W2, m_b2, m_W3, m_b3, m_W4, m_b4):
    batch = batch.astype(jnp.int32)
    brow = batch.reshape(_N, 1)
    bcol = batch.reshape(1, _N)

    xf1 = jnp.pad(x, ((0, 0), (0, 7)))
    wp1 = jnp.pad(c1_W1[0:1] - c1_W1[1:2], ((0, 7), (0, 0)))
    wb1 = jnp.pad(c1_W1[1:2], ((0, 7), (0, 0)))
    x1 = _edge_conv(xf1, xf1.T, brow, bcol, wp1, wb1,
                    c1_b1.reshape(1, _DH), c1_g1.reshape(1, _DH),
                    c1_be1.reshape(1, _DH), c1_W2, c1_b2.reshape(1, _DH))

    x2 = _edge_conv(x1, x1.T, brow, bcol,
                    c2_W1[:_DH] - c2_W1[_DH:], c2_W1[_DH:],
                    c2_b1.reshape(1, _DH), c2_g1.reshape(1, _DH),
                    c2_be1.reshape(1, _DH), c2_W2, c2_b2.reshape(1, _DH))

    x3 = _edge_conv(x2, x2.T, brow, bcol,
                    c3_W1[:_DH] - c3_W1[_DH:], c3_W1[_DH:],
                    c3_b1.reshape(1, _DH), c3_g1.reshape(1, _DH),
                    c3_be1.reshape(1, _DH), c3_W2, c3_b2.reshape(1, _DH))

    xc = jnp.concatenate([x1, x2, x3], axis=1)
    rb = 1024
    out = pl.pallas_call(
        _mlp_body,
        grid=(_N // rb,),
        in_specs=[pl.BlockSpec((rb, 192), lambda i: (i, 0))] + [
            pl.BlockSpec(s, lambda i: (0, 0)) for s in
            [(192, 256), (1, 256), (256, 128), (1, 128),
             (128, 64), (1, 64), (64, 16), (1, 16)]
        ],
        out_specs=pl.BlockSpec((rb, 16), lambda i: (i, 0)),
        out_shape=jax.ShapeDtypeStruct((_N, 16), jnp.float32),
    )(xc, m_W1, m_b1.reshape(1, 256), m_W2, m_b2.reshape(1, 128),
      m_W3, m_b3.reshape(1, 64), m_W4, m_b4.reshape(1, 16))
    return out


# speedup vs baseline: 8.3529x; 1.0305x over previous
import functools

import jax
import jax.numpy as jnp
from jax import lax
from jax.experimental import pallas as pl
from jax.experimental.pallas import tpu as pltpu
from jax.experimental.pallas import tpu_sc as plsc

_N = 4096
_K = 16
_NK = _N * _K
_EPS = 1e-5
_DH = 64
_ROWS = 256
_W = 1536
_W2 = 2560
_NW = 32
_CH = 128


_FBIG = 1.7014118346046923e38
_FBIG2 = 1.70141e38
_FMAX = 3.4028234663852886e38


def _scan_topk(key, r):
    kiota = lax.broadcasted_iota(jnp.int32, (r, _K), 1)
    idx_acc = jnp.zeros((r, _K), jnp.int32)
    m = jnp.min(key, axis=1, keepdims=True)
    for k in range(_K):
        mi = lax.bitcast_convert_type(m, jnp.int32) & 0xFFF
        idx_acc = jnp.where(kiota == k, mi, idx_acc)
        if k + 1 < _K:
            m = jnp.min(jnp.where(key > m, key, _FMAX), axis=1,
                        keepdims=True)
    return idx_acc, m


def _topk_from_key(key_i, r):
    key = lax.bitcast_convert_type(key_i, jnp.float32)
    w = key.shape[1]
    if w <= 512:
        return _scan_topk(key, r)[0]
    m1 = key[:, 0:128]
    m2 = jnp.full((r, 128), _FMAX, jnp.float32)
    m3 = m2
    m4 = m2
    for cch in range(1, w // 128):
        x = key[:, cch * 128:(cch + 1) * 128]
        t = jnp.minimum(m1, x); x = jnp.maximum(m1, x); m1 = t
        t = jnp.minimum(m2, x); x = jnp.maximum(m2, x); m2 = t
        t = jnp.minimum(m3, x); x = jnp.maximum(m3, x); m3 = t
        m4 = jnp.minimum(m4, x)
    cand = jnp.concatenate([m1, m2, m3, m4], axis=1)
    idx_fast, m16 = _scan_topk(cand, r)
    cnt = jnp.sum(jnp.where(key <= m16, 1, 0), axis=1)
    allok = jnp.all(cnt == _K)
    return lax.cond(allok,
                    lambda u: idx_fast,
                    lambda u: _scan_topk(key, r)[0], 0)


def _knn_pq_body(xf_ref, xaT_ref, brow_ref, bcol_ref, wp_ref, wb_ref, b1_ref,
                 idx_ref, p_ref, q_ref):
    xb = xf_ref[...]
    r = xb.shape[0]
    sqb = jnp.sum(xb * xb, axis=1, keepdims=True)
    brow = brow_ref[...]

    def compute_key(xaT, bcol, base):
        prod = lax.dot_general(xb, xaT, (((1,), (0,)), ((), ())),
                               preferred_element_type=jnp.float32)
        sqa = jnp.sum(xaT * xaT, axis=0, keepdims=True)
        d = (sqb + sqa) - 2.0 * prod
        d = jnp.minimum(jnp.maximum(d, 1.1754944e-38), _FBIG2)
        d = jnp.where(brow != bcol, _FBIG, d)
        iota = lax.broadcasted_iota(jnp.int32, d.shape, 1) + base
        return (lax.bitcast_convert_type(d, jnp.int32)
                & jnp.int32(~0xFFF)) | iota

    b_first = brow_ref[0, 0]
    b_last = brow_ref[r - 1, 0]
    bcol_full = bcol_ref[...]
    citer = lax.broadcasted_iota(jnp.int32, (1, _N), 1)
    lo = jnp.min(jnp.where(bcol_full == b_first, citer, _N))
    hi = jnp.max(jnp.where(bcol_full == b_last, citer, -1)) + 1
    lo128 = (lo // 128) * 128
    lo_a = jnp.minimum(lo128, _N - _W)
    lo_b = jnp.minimum(lo128, _N - _W2)

    def wpath(_):
        return _topk_from_key(
            compute_key(xaT_ref[:, pl.ds(lo_a, _W)],
                        bcol_ref[:, pl.ds(lo_a, _W)], lo_a), r)

    def mpath(_):
        return _topk_from_key(
            compute_key(xaT_ref[:, pl.ds(lo_b, _W2)],
                        bcol_ref[:, pl.ds(lo_b, _W2)], lo_b), r)

    def fpath(_):
        return _topk_from_key(compute_key(xaT_ref[...], bcol_full, 0), r)

    minsz = jnp.int32(_N)
    for g in range(4):
        sz = jnp.sum(jnp.where(bcol_full == g, 1, 0))
        ing = (g >= b_first) & (g <= b_last)
        minsz = jnp.where(ing, jnp.minimum(minsz, sz), minsz)
    ok = minsz >= _K

    idx_ref[...] = lax.cond(
        ((hi - lo_a) <= _W) & ok, wpath,
        lambda u: lax.cond(((hi - lo_b) <= _W2) & ok, mpath, fpath, u), 0)
    p_ref[...] = xb @ wp_ref[...] + b1_ref[...]
    q_ref[...] = xb @ wb_ref[...]


def _stats_body(qg_ref, p_ref, acc_ref):
    p = p_ref[...]
    s = jnp.zeros((1, _DH), jnp.float32)
    s2 = jnp.zeros((1, _DH), jnp.float32)
    for k in range(_K):
        h = qg_ref[k, :, :_DH] + p
        s = s + jnp.sum(h, axis=0, keepdims=True)
        s2 = s2 + jnp.sum(h * h, axis=0, keepdims=True)
    val = jnp.concatenate([s, s2], axis=0)
    @pl.when(pl.program_id(0) == 0)
    def _():
        acc_ref[...] = val
    @pl.when(pl.program_id(0) != 0)
    def _():
        acc_ref[...] = acc_ref[...] + val


def _edge_out_body(qg_ref, p_ref, acc_ref, g_ref, be_ref, w2_ref, b2_ref,
                   o_ref):
    inv_n = 1.0 / float(_NK)
    s = acc_ref[0:1, :]
    s2 = acc_ref[1:2, :]
    mean = s * inv_n
    var = s2 * inv_n - mean * mean
    a = g_ref[...] * lax.rsqrt(var + _EPS)
    c = be_ref[...] - a * mean
    p = p_ref[...]
    w2 = w2_ref[...]
    b2 = b2_ref[...]
    acc = jnp.full((_ROWS, _DH), -_FMAX, jnp.float32)
    for k in range(_K):
        h = qg_ref[k, :, :_DH] + p
        hn = jnp.maximum(h * a + c, 0.0)
        acc = jnp.maximum(acc, hn @ w2 + b2)
    o_ref[...] = acc


def _mlp_body(xc_ref, w1_ref, b1_ref, w2_ref, b2_ref, w3_ref, b3_ref,
              w4_ref, b4_ref, o_ref):
    h = xc_ref[...]
    h = jnp.maximum(h @ w1_ref[...] + b1_ref[...], 0.0)
    h = jnp.maximum(h @ w2_ref[...] + b2_ref[...], 0.0)
    h = jnp.maximum(h @ w3_ref[...] + b3_ref[...], 0.0)
    h = h @ w4_ref[...] + b4_ref[...]
    z = h - jnp.max(h, axis=1, keepdims=True)
    o_ref[...] = z - jnp.log(jnp.sum(jnp.exp(z), axis=1, keepdims=True))


def _gather_rows(q, idx_flat):
    per_w = _NK // _NW
    mesh = plsc.VectorSubcoreMesh(core_axis_name="c", subcore_axis_name="s")

    n_ch = per_w // _CH

    @functools.partial(
        pl.kernel,
        out_type=jax.ShapeDtypeStruct((_NK, 2 * _DH), jnp.float32),
        mesh=mesh,
        scratch_types=[
            pltpu.VMEM((per_w,), jnp.int32),
            pltpu.VMEM((_CH, 2 * _DH), jnp.float32),
            pltpu.VMEM((_CH, 2 * _DH), jnp.float32),
            pltpu.SemaphoreType.DMA,
            pltpu.SemaphoreType.DMA,
        ],
    )
    def gk(q_hbm, idx_hbm, out_hbm, idx_v, buf0, buf1, sem0, sem1):
        w = lax.axis_index("s") * 2 + lax.axis_index("c")
        base = w * per_w
        pltpu.sync_copy(idx_hbm.at[pl.ds(base, per_w)], idx_v)
        bufs = (buf0, buf1)
        sems = (sem0, sem1)
        copies = [None] * n_ch
        copies[0] = pltpu.async_copy(
            q_hbm.at[idx_v.at[pl.ds(0, _CH)]], bufs[0], sems[0])
        for cc in range(n_ch):
            if cc + 1 < n_ch:
                copies[cc + 1] = pltpu.async_copy(
                    q_hbm.at[idx_v.at[pl.ds((cc + 1) * _CH, _CH)]],
                    bufs[(cc + 1) % 2], sems[(cc + 1) % 2])
            copies[cc].wait()
            pltpu.sync_copy(bufs[cc % 2],
                            out_hbm.at[pl.ds(base + cc * _CH, _CH)])

    return gk(q, idx_flat)


def _edge_conv(xf, xaT, brow, bcol, wp, wb, b1, g1, be1, w2, b2):
    f = xf.shape[1]
    wb = jnp.pad(wb, ((0, 0), (0, _DH)))
    grid = (_N // _ROWS,)
    idx, p, q = pl.pallas_call(
        _knn_pq_body,
        grid=grid,
        in_specs=[
            pl.BlockSpec((_ROWS, f), lambda i: (i, 0)),
            pl.BlockSpec((f, _N), lambda i: (0, 0)),
            pl.BlockSpec((_ROWS, 1), lambda i: (i, 0)),
            pl.BlockSpec((1, _N), lambda i: (0, 0)),
            pl.BlockSpec((f, _DH), lambda i: (0, 0)),
            pl.BlockSpec((f, 2 * _DH), lambda i: (0, 0)),
            pl.BlockSpec((1, _DH), lambda i: (0, 0)),
        ],
        out_specs=[
            pl.BlockSpec((_ROWS, _K), lambda i: (i, 0)),
            pl.BlockSpec((_ROWS, _DH), lambda i: (i, 0)),
            pl.BlockSpec((_ROWS, 2 * _DH), lambda i: (i, 0)),
        ],
        out_shape=[
            jax.ShapeDtypeStruct((_N, _K), jnp.int32),
            jax.ShapeDtypeStruct((_N, _DH), jnp.float32),
            jax.ShapeDtypeStruct((_N, 2 * _DH), jnp.float32),
        ],
    )(xf, xaT, brow, bcol, wp, wb, b1)

    qg = _gather_rows(q, idx.T.reshape(_NK))
    qg3 = qg.reshape(_K, _N, 2 * _DH)

    acc = pl.pallas_call(
        _stats_body,
        grid=grid,
        in_specs=[
            pl.BlockSpec((_K, _ROWS, 2 * _DH), lambda i: (0, i, 0)),
            pl.BlockSpec((_ROWS, _DH), lambda i: (i, 0)),
        ],
        out_specs=pl.BlockSpec((2, _DH), lambda i: (0, 0)),
        out_shape=jax.ShapeDtypeStruct((2, _DH), jnp.float32),
        compiler_params=pltpu.CompilerParams(
            dimension_semantics=("arbitrary",)),
    )(qg3, p)

    xo = pl.pallas_call(
        _edge_out_body,
        grid=grid,
        in_specs=[
            pl.BlockSpec((_K, _ROWS, 2 * _DH), lambda i: (0, i, 0)),
            pl.BlockSpec((_ROWS, _DH), lambda i: (i, 0)),
            pl.BlockSpec((2, _DH), lambda i: (0, 0)),
            pl.BlockSpec((1, _DH), lambda i: (0, 0)),
            pl.BlockSpec((1, _DH), lambda i: (0, 0)),
            pl.BlockSpec((_DH, _DH), lambda i: (0, 0)),
            pl.BlockSpec((1, _DH), lambda i: (0, 0)),
        ],
        out_specs=pl.BlockSpec((_ROWS, _DH), lambda i: (i, 0)),
        out_shape=jax.ShapeDtypeStruct((_N, _DH), jnp.float32),
    )(qg3, p, acc, g1, be1, w2, b2)
    return xo


def kernel(x, batch, c1_W1, c1_b1, c1_g1, c1_be1, c1_W2, c1_b2,
           c2_W1, c2_b1, c2_g1, c2_be1, c2_W2, c2_b2,
           c3_W1, c3_b1, c3_g1, c3_be1, c3_W2, c3_b2,
           m_W1, m_b1, m_W2, m_b2, m_W3, m_b3, m_W4, m_b4):
    batch = batch.astype(jnp.int32)
    brow = batch.reshape(_N, 1)
    bcol = batch.reshape(1, _N)

    xf1 = jnp.pad(x, ((0, 0), (0, 7)))
    wp1 = jnp.pad(c1_W1[0:1] - c1_W1[1:2], ((0, 7), (0, 0)))
    wb1 = jnp.pad(c1_W1[1:2], ((0, 7), (0, 0)))
    x1 = _edge_conv(xf1, xf1.T, brow, bcol, wp1, wb1,
                    c1_b1.reshape(1, _DH), c1_g1.reshape(1, _DH),
                    c1_be1.reshape(1, _DH), c1_W2, c1_b2.reshape(1, _DH))

    x2 = _edge_conv(x1, x1.T, brow, bcol,
                    c2_W1[:_DH] - c2_W1[_DH:], c2_W1[_DH:],
                    c2_b1.reshape(1, _DH), c2_g1.reshape(1, _DH),
                    c2_be1.reshape(1, _DH), c2_W2, c2_b2.reshape(1, _DH))

    x3 = _edge_conv(x2, x2.T, brow, bcol,
                    c3_W1[:_DH] - c3_W1[_DH:], c3_W1[_DH:],
                    c3_b1.reshape(1, _DH), c3_g1.reshape(1, _DH),
                    c3_be1.reshape(1, _DH), c3_W2, c3_b2.reshape(1, _DH))

    xc = jnp.concatenate([x1, x2, x3], axis=1)
    rb = 1024
    out = pl.pallas_call(
        _mlp_body,
        grid=(_N // rb,),
        in_specs=[pl.BlockSpec((rb, 192), lambda i: (i, 0))] + [
            pl.BlockSpec(s, lambda i: (0, 0)) for s in
            [(192, 256), (1, 256), (256, 128), (1, 128),
             (128, 64), (1, 64), (64, 16), (1, 16)]
        ],
        out_specs=pl.BlockSpec((rb, 16), lambda i: (i, 0)),
        out_shape=jax.ShapeDtypeStruct((_N, 16), jnp.float32),
    )(xc, m_W1, m_b1.reshape(1, 256), m_W2, m_b2.reshape(1, 128),
      m_W3, m_b3.reshape(1, 64), m_W4, m_b4.reshape(1, 16))
    return out
